# baseline (speedup 1.0000x reference)
import functools

import jax
import jax.numpy as jnp
from jax import lax
from jax.experimental import pallas as pl
from jax.experimental.pallas import tpu as pltpu
from jax.experimental.pallas import tpu_sc as plsc

_TT = 16
_NBUF = 4

_NW = 32


def _sc_gather_kernel(idx_hbm, exttab_hbm, out_hbm, idx_v, rows_v, gsem,
                      osem):
    rpw = rows_v.shape[1]
    wid = lax.axis_index("s") * 2 + lax.axis_index("c")
    r0 = wid * 2 * rpw

    pltpu.sync_copy(idx_hbm.at[pl.ds(r0, 2 * rpw)], idx_v)
    g0 = pltpu.make_async_copy(
        exttab_hbm.at[idx_v.at[pl.ds(0, rpw)]], rows_v.at[0], gsem)
    g1 = pltpu.make_async_copy(
        exttab_hbm.at[idx_v.at[pl.ds(rpw, rpw)]], rows_v.at[1], osem)
    g0.start()
    g1.start()
    g0.wait()
    o0 = pltpu.make_async_copy(
        rows_v.at[0], out_hbm.at[pl.ds(r0, rpw)], gsem)
    o0.start()
    g1.wait()
    o1 = pltpu.make_async_copy(
        rows_v.at[1], out_hbm.at[pl.ds(r0 + rpw, rpw)], osem)
    o1.start()
    o0.wait()
    o1.wait()


def _tc_kernel(eids_ref, regrows_ref, spikes_ref, wfull_ref, bcol_ref,
               slott_ref, eidembt_ref, out_ref, base_ref, obuf_ref, sems):
    b_idx = pl.program_id(0)
    t_idx = pl.program_id(1)
    nt = pl.num_programs(1)
    nsteps = pl.num_programs(0) * nt
    i = b_idx * nt + t_idx
    slot = jax.lax.rem(i, _NBUF)
    tt = obuf_ref.shape[1]
    d, n = base_ref.shape

    @pl.when(t_idx == 0)
    def _build_base():
        regt = regrows_ref[...][:, 0:d].T
        e = eids_ref[b_idx]
        neids = eidembt_ref.shape[1]
        ohe = (jax.lax.broadcasted_iota(jnp.int32, (neids, 8), 0) == e
               ).astype(jnp.float32)
        evt = jnp.dot(eidembt_ref[...], ohe,
                      preferred_element_type=jnp.float32)
        base_ref[...] = (slott_ref[...] + regt
                         + evt[:, 0:1] + bcol_ref[...])

    dst = out_ref.at[b_idx, pl.ds(t_idx * tt, tt)]

    @pl.when(i >= _NBUF)
    def _wait_slot():
        pltpu.make_async_copy(obuf_ref.at[slot], dst, sems.at[slot]).wait()

    sp = spikes_ref[0]
    obuf_ref[slot] = (sp[:, None, :] * wfull_ref[...][None, :, :]
                      + base_ref[...][None, :, :])
    pltpu.make_async_copy(obuf_ref.at[slot], dst, sems.at[slot]).start()

    @pl.when(i == nsteps - 1)
    def _drain():
        for k in range(_NBUF):
            pltpu.make_async_copy(obuf_ref.at[k], dst, sems.at[k]).wait()


@jax.jit
def kernel(spikes, neuron_regions, eids, w_spike, b_spike, neuron_slot,
           region_emb, eid_emb):
    B, T, N = spikes.shape
    D = neuron_slot.shape[1]
    rows = B * N
    rpw = rows // _NW

    eids32 = eids.astype(jnp.int32)
    idxflat = neuron_regions.astype(jnp.int32).reshape(-1)
    exttab = jnp.pad(region_emb, ((0, 0), (0, 128 - D)))

    scmesh = plsc.VectorSubcoreMesh(core_axis_name="c", subcore_axis_name="s")
    sc_gather = functools.partial(
        pl.kernel,
        out_type=jax.ShapeDtypeStruct((rows, 128), jnp.float32),
        mesh=scmesh,
        scratch_types=[
            pltpu.VMEM((rpw,), jnp.int32),
            pltpu.VMEM((2, rpw // 2, 128), jnp.float32),
            pltpu.SemaphoreType.DMA,
            pltpu.SemaphoreType.DMA,
        ],
    )(_sc_gather_kernel)
    regrows = sc_gather(idxflat, exttab)

    wfull = jnp.broadcast_to(w_spike, (D, N))
    bcol = b_spike.reshape(D, 1)
    slott = neuron_slot[:N].T
    eidembt = eid_emb.T

    outt = pl.pallas_call(
        _tc_kernel,
        grid=(B, T // _TT),
        in_specs=[
            pl.BlockSpec(memory_space=pltpu.SMEM),
            pl.BlockSpec((N, 128), lambda b, t: (b, 0)),
            pl.BlockSpec((1, _TT, N), lambda b, t: (b, t, 0)),
            pl.BlockSpec((D, N), lambda b, t: (0, 0)),
            pl.BlockSpec((D, 1), lambda b, t: (0, 0)),
            pl.BlockSpec((D, N), lambda b, t: (0, 0)),
            pl.BlockSpec((D, eid_emb.shape[0]), lambda b, t: (0, 0)),
        ],
        out_specs=pl.BlockSpec(memory_space=pltpu.MemorySpace.HBM),
        out_shape=jax.ShapeDtypeStruct((B, T, D, N), jnp.float32),
        scratch_shapes=[
            pltpu.VMEM((D, N), jnp.float32),
            pltpu.VMEM((_NBUF, _TT, D, N), jnp.float32),
            pltpu.SemaphoreType.DMA((_NBUF,)),
        ],
    )(eids32, regrows, spikes, wfull, bcol, slott, eidembt)
    return outt.transpose(0, 1, 3, 2)
